# Initial kernel scaffold; baseline (speedup 1.0000x reference)
import jax
import jax.numpy as jnp
from jax.experimental import pallas as pl


def kernel(x, edge_index, W1, b1, W2, b2):
    raise NotImplementedError("write your pallas kernel here")


# speedup vs baseline: 21.9612x; 21.9612x over previous
import functools

import jax
import jax.numpy as jnp
from jax import lax
from jax.experimental import pallas as pl
from jax.experimental.pallas import tpu as pltpu
from jax.experimental.pallas import tpu_sc as plsc

NC = 2
NS = 16
NW = NC * NS
CHUNK = 128


def _mesh():
    return plsc.VectorSubcoreMesh(core_axis_name="c", subcore_axis_name="s")


_SC_PARAMS = pltpu.CompilerParams(use_tc_tiling_on_sc=False)


def _make_deg(E, NPAD):
    EPW = E // NW
    NFULL = EPW // CHUNK
    TAIL = EPW - NFULL * CHUNK
    RPW = NPAD // NS

    @functools.partial(
        pl.kernel,
        out_type=jax.ShapeDtypeStruct((NC * NPAD,), jnp.float32),
        mesh=_mesh(),
        compiler_params=_SC_PARAMS,
        scratch_types=[
            pltpu.VMEM((CHUNK,), jnp.int32),
            pltpu.VMEM((CHUNK,), jnp.float32),
            pltpu.VMEM((max(TAIL, 8),), jnp.int32),
            pltpu.VMEM((max(TAIL, 8),), jnp.float32),
            pltpu.VMEM((RPW,), jnp.float32),
            pltpu.VMEM_SHARED((NPAD,), jnp.float32),
        ],
    )
    def deg_kernel(dst_hbm, ones_hbm, zeros_hbm, out_hbm,
                   idx_v, ones_v, idx_t, ones_t, stage, acc):
        c = lax.axis_index("c")
        s = lax.axis_index("s")
        wid = c * NS + s
        r0 = s * RPW
        pltpu.sync_copy(zeros_hbm.at[pl.ds(r0, RPW)], stage)
        pltpu.sync_copy(stage, acc.at[pl.ds(r0, RPW)])
        pltpu.sync_copy(ones_hbm, ones_v)
        if TAIL:
            pltpu.sync_copy(ones_hbm.at[pl.ds(0, TAIL)],
                            ones_t.at[pl.ds(0, TAIL)])
        plsc.subcore_barrier()

        def body(i, carry):
            base = wid * EPW + i * CHUNK
            pltpu.sync_copy(dst_hbm.at[pl.ds(base, CHUNK)], idx_v)
            pltpu.sync_copy(ones_v, acc.at[idx_v], add=True)
            return carry

        lax.fori_loop(0, NFULL, body, 0)
        if TAIL:
            base = wid * EPW + NFULL * CHUNK
            pltpu.sync_copy(dst_hbm.at[pl.ds(base, TAIL)],
                            idx_t.at[pl.ds(0, TAIL)])
            pltpu.sync_copy(ones_t.at[pl.ds(0, TAIL)],
                            acc.at[idx_t.at[pl.ds(0, TAIL)]], add=True)
        plsc.subcore_barrier()
        pltpu.sync_copy(acc.at[pl.ds(r0, RPW)], stage)
        pltpu.sync_copy(stage, out_hbm.at[pl.ds(c * NPAD + r0, RPW)])

    return deg_kernel


def _make_agg(E, N, NPAD, D):
    EPW = E // NW
    NFULL = EPW // CHUNK
    TAIL = EPW - NFULL * CHUNK
    RPW = NPAD // NS

    @functools.partial(
        pl.kernel,
        out_type=jax.ShapeDtypeStruct((NC, NPAD, D), jnp.float32),
        mesh=_mesh(),
        compiler_params=_SC_PARAMS,
        scratch_types=[
            pltpu.VMEM((CHUNK,), jnp.int32),
            pltpu.VMEM((CHUNK,), jnp.int32),
            pltpu.VMEM((CHUNK, D), jnp.float32),
            pltpu.VMEM((max(TAIL, 8),), jnp.int32),
            pltpu.VMEM((max(TAIL, 8),), jnp.int32),
            pltpu.VMEM((max(TAIL, 8), D), jnp.float32),
            pltpu.VMEM((RPW, D), jnp.float32),
            pltpu.VMEM_SHARED((NPAD, D), jnp.float32),
            pltpu.SemaphoreType.DMA,
        ],
    )
    def agg_kernel(src_hbm, dst_hbm, h_hbm, zeros_hbm, out_hbm,
                   sidx, didx, rows, sidx_t, didx_t, rows_t, stage, acc, sem):
        c = lax.axis_index("c")
        s = lax.axis_index("s")
        wid = c * NS + s
        r0 = s * RPW
        pltpu.sync_copy(zeros_hbm.at[pl.ds(r0, RPW)], stage)
        pltpu.sync_copy(stage, acc.at[pl.ds(r0, RPW)])
        plsc.subcore_barrier()

        def body(i, carry):
            base = wid * EPW + i * CHUNK
            pltpu.sync_copy(src_hbm.at[pl.ds(base, CHUNK)], sidx)
            pltpu.sync_copy(dst_hbm.at[pl.ds(base, CHUNK)], didx)
            pltpu.async_copy(h_hbm.at[sidx], rows, sem).wait()
            pltpu.sync_copy(rows, acc.at[didx], add=True)
            return carry

        lax.fori_loop(0, NFULL, body, 0)
        if TAIL:
            base = wid * EPW + NFULL * CHUNK
            pltpu.sync_copy(src_hbm.at[pl.ds(base, TAIL)],
                            sidx_t.at[pl.ds(0, TAIL)])
            pltpu.sync_copy(dst_hbm.at[pl.ds(base, TAIL)],
                            didx_t.at[pl.ds(0, TAIL)])
            pltpu.async_copy(h_hbm.at[sidx_t.at[pl.ds(0, TAIL)]],
                             rows_t.at[pl.ds(0, TAIL)], sem).wait()
            pltpu.sync_copy(rows_t.at[pl.ds(0, TAIL)],
                            acc.at[didx_t.at[pl.ds(0, TAIL)]], add=True)
        plsc.subcore_barrier()
        pltpu.sync_copy(acc.at[pl.ds(r0, RPW)], stage)
        pltpu.sync_copy(stage, out_hbm.at[c, pl.ds(r0, RPW)])

    return agg_kernel


def _tc1(x, W1, degt, N):
    DH = W1.shape[1]

    def body(x_ref, w_ref, degt_ref, hs_ref, dis_ref):
        degt = degt_ref[...]
        deg = degt[:N, 0:1] + degt[:N, 1:2] + 1.0
        dis = lax.rsqrt(deg)
        h = jnp.dot(x_ref[...], w_ref[...], preferred_element_type=jnp.float32)
        hs_ref[...] = h * dis
        dis_ref[...] = dis

    return pl.pallas_call(
        body,
        out_shape=[
            jax.ShapeDtypeStruct((N, DH), jnp.float32),
            jax.ShapeDtypeStruct((N, 1), jnp.float32),
        ],
    )(x, W1, degt)


def _tc2(aggp, hs, dis, b1, W2p, N):
    DP = W2p.shape[1]

    def body(aggp_ref, hs_ref, dis_ref, b1_ref, w_ref, out_ref):
        agg = aggp_ref[...]
        tot = (agg[0, :N, :] + agg[1, :N, :] + hs_ref[...]) * dis_ref[...]
        z = jnp.maximum(tot + b1_ref[...], 0.0)
        h2 = jnp.dot(z, w_ref[...], preferred_element_type=jnp.float32)
        out_ref[...] = h2 * dis_ref[...]

    return pl.pallas_call(
        body,
        out_shape=jax.ShapeDtypeStruct((N, DP), jnp.float32),
    )(aggp, hs, dis, b1, W2p)


def _tc3(aggp, h2s, dis, b2p, N, DO):
    def body(aggp_ref, h2s_ref, dis_ref, b2_ref, out_ref):
        agg = aggp_ref[...]
        o = (agg[0, :N, :] + agg[1, :N, :] + h2s_ref[...]) * dis_ref[...]
        o = o + b2_ref[...]
        o7 = o[:, :DO]
        m = jnp.max(o7, axis=1, keepdims=True)
        ex = jnp.exp(o7 - m)
        lse = jnp.log(jnp.sum(ex, axis=1, keepdims=True)) + m
        out_ref[...] = o7 - lse

    return pl.pallas_call(
        body,
        out_shape=jax.ShapeDtypeStruct((N, DO), jnp.float32),
    )(aggp, h2s, dis, b2p)


def kernel(x, edge_index, W1, b1, W2, b2):
    N, _ = x.shape
    DH = W1.shape[1]
    DO = W2.shape[1]
    E = edge_index.shape[1]
    DP = 8

    NPAD = -(-N // (NS * 8)) * (NS * 8)

    src = edge_index[0]
    dst = edge_index[1]
    ones = jnp.ones((CHUNK,), jnp.float32)
    z1 = jnp.zeros((NPAD,), jnp.float32)
    z16 = jnp.zeros((NPAD, DH), jnp.float32)
    z8 = jnp.zeros((NPAD, DP), jnp.float32)
    W2p = jnp.zeros((DH, DP), jnp.float32).at[:, :DO].set(W2)
    b1r = b1.reshape(1, DH)
    b2p = jnp.zeros((1, DP), jnp.float32).at[0, :DO].set(b2)

    degp = _make_deg(E, NPAD)(dst, ones, z1)
    degt = degp.reshape(NC, NPAD).T
    hs, dis = _tc1(x, W1, degt, N)
    aggp1 = _make_agg(E, N, NPAD, DH)(src, dst, hs, z16)
    h2s = _tc2(aggp1, hs, dis, b1r, W2p, N)
    aggp2 = _make_agg(E, N, NPAD, DP)(src, dst, h2s, z8)
    return _tc3(aggp2, h2s, dis, b2p, N, DO)


# speedup vs baseline: 57.0864x; 2.5994x over previous
import functools

import jax
import jax.numpy as jnp
from jax import lax
from jax.experimental import pallas as pl
from jax.experimental.pallas import tpu as pltpu
from jax.experimental.pallas import tpu_sc as plsc

NC = 2
NS = 16
NW = NC * NS
CHUNK = 128


def _mesh():
    return plsc.VectorSubcoreMesh(core_axis_name="c", subcore_axis_name="s")


_SC_PARAMS = pltpu.CompilerParams(use_tc_tiling_on_sc=False)


def _group_k(nfull):
    for k in range(16, 0, -1):
        if nfull % k == 0:
            return k
    return 1


def _make_deg(E, NPAD):
    EPW = E // NW
    NFULL = EPW // CHUNK
    TAIL = EPW - NFULL * CHUNK
    RPW = NPAD // NS
    K = _group_k(NFULL)
    NG = NFULL // K

    @functools.partial(
        pl.kernel,
        out_type=jax.ShapeDtypeStruct((NC * NPAD,), jnp.float32),
        mesh=_mesh(),
        compiler_params=_SC_PARAMS,
        scratch_types=[
            pltpu.VMEM((NFULL, CHUNK), jnp.int32),
            pltpu.VMEM((CHUNK,), jnp.float32),
            pltpu.VMEM((max(TAIL, 8),), jnp.int32),
            pltpu.VMEM((max(TAIL, 8),), jnp.float32),
            pltpu.VMEM((RPW,), jnp.float32),
            pltpu.VMEM_SHARED((NPAD,), jnp.float32),
            pltpu.SemaphoreType.DMA,
        ],
    )
    def deg_kernel(dstm_hbm, dstt_hbm, ones_hbm, zeros_hbm, out_hbm,
                   idx_v, ones_v, idx_t, ones_t, stage, acc, sem):
        c = lax.axis_index("c")
        s = lax.axis_index("s")
        wid = c * NS + s
        r0 = s * RPW
        pltpu.sync_copy(zeros_hbm.at[pl.ds(r0, RPW)], stage)
        pltpu.sync_copy(stage, acc.at[pl.ds(r0, RPW)])
        pltpu.sync_copy(dstm_hbm.at[wid], idx_v)
        pltpu.sync_copy(ones_hbm, ones_v)
        if TAIL:
            pltpu.sync_copy(dstt_hbm.at[wid], idx_t.at[pl.ds(0, TAIL)])
            pltpu.sync_copy(ones_hbm.at[pl.ds(0, TAIL)],
                            ones_t.at[pl.ds(0, TAIL)])
        plsc.subcore_barrier()

        def body(j, carry):
            ds = [pltpu.async_copy(ones_v.at[pl.ds(0, CHUNK)],
                                   acc.at[idx_v.at[j * K + b]], sem,
                                   add=True)
                  for b in range(K)]
            for d in ds:
                d.wait()
            return carry

        lax.fori_loop(0, NG, body, 0)
        if TAIL:
            pltpu.sync_copy(ones_t.at[pl.ds(0, TAIL)],
                            acc.at[idx_t.at[pl.ds(0, TAIL)]], add=True)
        plsc.subcore_barrier()
        pltpu.sync_copy(acc.at[pl.ds(r0, RPW)], stage)
        pltpu.sync_copy(stage, out_hbm.at[pl.ds(c * NPAD + r0, RPW)])

    return deg_kernel


def _make_agg(E, N, NPAD, D):
    EPW = E // NW
    NFULL = EPW // CHUNK
    TAIL = EPW - NFULL * CHUNK
    RPW = NPAD // NS
    K = _group_k(NFULL)
    NG = NFULL // K

    @functools.partial(
        pl.kernel,
        out_type=jax.ShapeDtypeStruct((NC, NPAD, D), jnp.float32),
        mesh=_mesh(),
        compiler_params=_SC_PARAMS,
        scratch_types=[
            pltpu.VMEM((NFULL, CHUNK), jnp.int32),
            pltpu.VMEM((NFULL, CHUNK), jnp.int32),
            pltpu.VMEM((K, CHUNK, D), jnp.float32),
            pltpu.VMEM((max(TAIL, 8),), jnp.int32),
            pltpu.VMEM((max(TAIL, 8),), jnp.int32),
            pltpu.VMEM((max(TAIL, 8), D), jnp.float32),
            pltpu.VMEM((RPW, D), jnp.float32),
            pltpu.VMEM_SHARED((NPAD, D), jnp.float32),
            pltpu.SemaphoreType.DMA,
            pltpu.SemaphoreType.DMA,
        ],
    )
    def agg_kernel(srcm_hbm, dstm_hbm, srct_hbm, dstt_hbm, h_hbm, zeros_hbm,
                   out_hbm, sidx, didx, rows, sidx_t, didx_t, rows_t, stage,
                   acc, semg, sems):
        c = lax.axis_index("c")
        s = lax.axis_index("s")
        wid = c * NS + s
        r0 = s * RPW
        pltpu.sync_copy(zeros_hbm.at[pl.ds(r0, RPW)], stage)
        pltpu.sync_copy(stage, acc.at[pl.ds(r0, RPW)])
        pltpu.sync_copy(srcm_hbm.at[wid], sidx)
        pltpu.sync_copy(dstm_hbm.at[wid], didx)
        if TAIL:
            pltpu.sync_copy(srct_hbm.at[wid], sidx_t.at[pl.ds(0, TAIL)])
            pltpu.sync_copy(dstt_hbm.at[wid], didx_t.at[pl.ds(0, TAIL)])
        plsc.subcore_barrier()

        def body(j, carry):
            gs = [pltpu.async_copy(h_hbm.at[sidx.at[j * K + b]],
                                   rows.at[b], semg)
                  for b in range(K)]
            for g in gs:
                g.wait()
            ss = [pltpu.async_copy(rows.at[b], acc.at[didx.at[j * K + b]],
                                   sems, add=True)
                  for b in range(K)]
            for d in ss:
                d.wait()
            return carry

        lax.fori_loop(0, NG, body, 0)
        if TAIL:
            pltpu.async_copy(h_hbm.at[sidx_t.at[pl.ds(0, TAIL)]],
                             rows_t.at[pl.ds(0, TAIL)], semg).wait()
            pltpu.sync_copy(rows_t.at[pl.ds(0, TAIL)],
                            acc.at[didx_t.at[pl.ds(0, TAIL)]], add=True)
        plsc.subcore_barrier()
        pltpu.sync_copy(acc.at[pl.ds(r0, RPW)], stage)
        pltpu.sync_copy(stage, out_hbm.at[c, pl.ds(r0, RPW)])

    return agg_kernel


def _tc1(x, W1, degt, N):
    DH = W1.shape[1]

    def body(x_ref, w_ref, degt_ref, hs_ref, dis_ref):
        degt = degt_ref[...]
        deg = degt[:N, 0:1] + degt[:N, 1:2] + 1.0
        dis = lax.rsqrt(deg)
        h = jnp.dot(x_ref[...], w_ref[...], preferred_element_type=jnp.float32)
        hs_ref[...] = h * dis
        dis_ref[...] = dis

    return pl.pallas_call(
        body,
        out_shape=[
            jax.ShapeDtypeStruct((N, DH), jnp.float32),
            jax.ShapeDtypeStruct((N, 1), jnp.float32),
        ],
    )(x, W1, degt)


def _tc2(aggp, hs, dis, b1, W2p, N):
    DP = W2p.shape[1]

    def body(aggp_ref, hs_ref, dis_ref, b1_ref, w_ref, out_ref):
        agg = aggp_ref[...]
        tot = (agg[0, :N, :] + agg[1, :N, :] + hs_ref[...]) * dis_ref[...]
        z = jnp.maximum(tot + b1_ref[...], 0.0)
        h2 = jnp.dot(z, w_ref[...], preferred_element_type=jnp.float32)
        out_ref[...] = h2 * dis_ref[...]

    return pl.pallas_call(
        body,
        out_shape=jax.ShapeDtypeStruct((N, DP), jnp.float32),
    )(aggp, hs, dis, b1, W2p)


def _tc3(aggp, h2s, dis, b2p, N, DO):
    def body(aggp_ref, h2s_ref, dis_ref, b2_ref, out_ref):
        agg = aggp_ref[...]
        o = (agg[0, :N, :] + agg[1, :N, :] + h2s_ref[...]) * dis_ref[...]
        o = o + b2_ref[...]
        o7 = o[:, :DO]
        m = jnp.max(o7, axis=1, keepdims=True)
        ex = jnp.exp(o7 - m)
        lse = jnp.log(jnp.sum(ex, axis=1, keepdims=True)) + m
        out_ref[...] = o7 - lse

    return pl.pallas_call(
        body,
        out_shape=jax.ShapeDtypeStruct((N, DO), jnp.float32),
    )(aggp, h2s, dis, b2p)


def kernel(x, edge_index, W1, b1, W2, b2):
    N, _ = x.shape
    DH = W1.shape[1]
    DO = W2.shape[1]
    E = edge_index.shape[1]
    DP = 8

    NPAD = -(-N // (NS * 8)) * (NS * 8)

    EPW = E // NW
    NFULL = EPW // CHUNK
    TAIL = EPW - NFULL * CHUNK
    src = edge_index[0].reshape(NW, EPW)
    dst = edge_index[1].reshape(NW, EPW)
    srcm = src[:, :NFULL * CHUNK].reshape(NW, NFULL, CHUNK)
    dstm = dst[:, :NFULL * CHUNK].reshape(NW, NFULL, CHUNK)
    srct = src[:, NFULL * CHUNK:]
    dstt = dst[:, NFULL * CHUNK:]
    ones = jnp.ones((CHUNK,), jnp.float32)
    z1 = jnp.zeros((NPAD,), jnp.float32)
    z16 = jnp.zeros((NPAD, DH), jnp.float32)
    z8 = jnp.zeros((NPAD, DP), jnp.float32)
    W2p = jnp.zeros((DH, DP), jnp.float32).at[:, :DO].set(W2)
    b1r = b1.reshape(1, DH)
    b2p = jnp.zeros((1, DP), jnp.float32).at[0, :DO].set(b2)

    degp = _make_deg(E, NPAD)(dstm, dstt, ones, z1)
    degt = degp.reshape(NC, NPAD).T
    hs, dis = _tc1(x, W1, degt, N)
    aggp1 = _make_agg(E, N, NPAD, DH)(srcm, dstm, srct, dstt, hs, z16)
    h2s = _tc2(aggp1, hs, dis, b1r, W2p, N)
    aggp2 = _make_agg(E, N, NPAD, DP)(srcm, dstm, srct, dstt, h2s, z8)
    return _tc3(aggp2, h2s, dis, b2p, N, DO)


# speedup vs baseline: 60.3892x; 1.0579x over previous
import functools

import jax
import jax.numpy as jnp
from jax import lax
from jax.experimental import pallas as pl
from jax.experimental.pallas import tpu as pltpu
from jax.experimental.pallas import tpu_sc as plsc

NC = 2
NS = 16
NW = NC * NS
CHUNK = 128


def _mesh():
    return plsc.VectorSubcoreMesh(core_axis_name="c", subcore_axis_name="s")


_SC_PARAMS = pltpu.CompilerParams(use_tc_tiling_on_sc=False)


def _group_k(nfull):
    for k in range(16, 0, -1):
        if nfull % k == 0:
            return k
    return 1


def _make_deg(E, NPAD):
    EPW = E // NW
    NFULL = EPW // CHUNK
    TAIL = EPW - NFULL * CHUNK
    RPW = NPAD // NS
    K = _group_k(NFULL)
    NG = NFULL // K

    @functools.partial(
        pl.kernel,
        out_type=jax.ShapeDtypeStruct((NC * NPAD,), jnp.float32),
        mesh=_mesh(),
        compiler_params=_SC_PARAMS,
        scratch_types=[
            pltpu.VMEM((NFULL, CHUNK), jnp.int32),
            pltpu.VMEM((CHUNK,), jnp.float32),
            pltpu.VMEM((max(TAIL, 8),), jnp.int32),
            pltpu.VMEM((max(TAIL, 8),), jnp.float32),
            pltpu.VMEM((RPW,), jnp.float32),
            pltpu.VMEM_SHARED((NPAD,), jnp.float32),
            pltpu.SemaphoreType.DMA,
        ],
    )
    def deg_kernel(dstm_hbm, dstt_hbm, ones_hbm, zeros_hbm, out_hbm,
                   idx_v, ones_v, idx_t, ones_t, stage, acc, sem):
        c = lax.axis_index("c")
        s = lax.axis_index("s")
        wid = c * NS + s
        r0 = s * RPW
        pltpu.sync_copy(zeros_hbm.at[pl.ds(r0, RPW)], stage)
        pltpu.sync_copy(stage, acc.at[pl.ds(r0, RPW)])
        pltpu.sync_copy(dstm_hbm.at[wid], idx_v)
        pltpu.sync_copy(ones_hbm, ones_v)
        if TAIL:
            pltpu.sync_copy(dstt_hbm.at[wid], idx_t.at[pl.ds(0, TAIL)])
            pltpu.sync_copy(ones_hbm.at[pl.ds(0, TAIL)],
                            ones_t.at[pl.ds(0, TAIL)])
        plsc.subcore_barrier()

        def body(j, carry):
            ds = [pltpu.async_copy(ones_v.at[pl.ds(0, CHUNK)],
                                   acc.at[idx_v.at[j * K + b]], sem,
                                   add=True)
                  for b in range(K)]
            for d in ds:
                d.wait()
            return carry

        lax.fori_loop(0, NG, body, 0)
        if TAIL:
            pltpu.sync_copy(ones_t.at[pl.ds(0, TAIL)],
                            acc.at[idx_t.at[pl.ds(0, TAIL)]], add=True)
        plsc.subcore_barrier()
        pltpu.sync_copy(acc.at[pl.ds(r0, RPW)], stage)
        pltpu.sync_copy(stage, out_hbm.at[pl.ds(c * NPAD + r0, RPW)])

    return deg_kernel


def _make_agg(E, N, NPAD, D):
    EPW = E // NW
    NFULL = EPW // CHUNK
    TAIL = EPW - NFULL * CHUNK
    RPW = NPAD // NS
    K = _group_k(NFULL)
    NG = NFULL // K

    @functools.partial(
        pl.kernel,
        out_type=jax.ShapeDtypeStruct((NC, NPAD, D), jnp.float32),
        mesh=_mesh(),
        compiler_params=_SC_PARAMS,
        scratch_types=[
            pltpu.VMEM((NFULL, CHUNK), jnp.int32),
            pltpu.VMEM((NFULL, CHUNK), jnp.int32),
            pltpu.VMEM((2, K, CHUNK, D), jnp.float32),
            pltpu.VMEM((max(TAIL, 8),), jnp.int32),
            pltpu.VMEM((max(TAIL, 8),), jnp.int32),
            pltpu.VMEM((max(TAIL, 8), D), jnp.float32),
            pltpu.VMEM((RPW, D), jnp.float32),
            pltpu.VMEM_SHARED((NPAD, D), jnp.float32),
            pltpu.SemaphoreType.DMA((2,)),
            pltpu.SemaphoreType.DMA((2,)),
        ],
    )
    def agg_kernel(srcm_hbm, dstm_hbm, srct_hbm, dstt_hbm, h_hbm, zeros_hbm,
                   out_hbm, sidx, didx, rows, sidx_t, didx_t, rows_t, stage,
                   acc, semg, sems):
        c = lax.axis_index("c")
        s = lax.axis_index("s")
        wid = c * NS + s
        r0 = s * RPW
        pltpu.sync_copy(zeros_hbm.at[pl.ds(r0, RPW)], stage)
        pltpu.sync_copy(stage, acc.at[pl.ds(r0, RPW)])
        pltpu.sync_copy(srcm_hbm.at[wid], sidx)
        pltpu.sync_copy(dstm_hbm.at[wid], didx)
        if TAIL:
            pltpu.sync_copy(srct_hbm.at[wid], sidx_t.at[pl.ds(0, TAIL)])
            pltpu.sync_copy(dstt_hbm.at[wid], didx_t.at[pl.ds(0, TAIL)])
        plsc.subcore_barrier()

        def fire_g(g, h):
            return [pltpu.async_copy(h_hbm.at[sidx.at[g * K + b]],
                                     rows.at[h, b], semg.at[h])
                    for b in range(K)]

        def fire_s(g, h):
            return [pltpu.async_copy(rows.at[h, b],
                                     acc.at[didx.at[g * K + b]],
                                     sems.at[h], add=True)
                    for b in range(K)]

        def drain(ds):
            for d in ds:
                d.wait()

        def drain_g(h):
            for b in range(K):
                pltpu.make_async_copy(h_hbm.at[sidx.at[b]],
                                      rows.at[h, b], semg.at[h]).wait()

        if NG % 2 == 0 and NG >= 2:
            fire_g(0, 0)

            def body(p, carry):
                ga = 2 * p
                drain_g(0)
                sa = fire_s(ga, 0)
                gb = fire_g(ga + 1, 1)
                drain(sa)

                @pl.when(p < NG // 2 - 1)
                def _():
                    fire_g(ga + 2, 0)
                drain(gb)
                drain(fire_s(ga + 1, 1))
                return carry

            lax.fori_loop(0, NG // 2, body, 0)
        else:
            def body1(j, carry):
                drain(fire_g(j, 0))
                drain(fire_s(j, 0))
                return carry

            lax.fori_loop(0, NG, body1, 0)
        if TAIL:
            pltpu.async_copy(h_hbm.at[sidx_t.at[pl.ds(0, TAIL)]],
                             rows_t.at[pl.ds(0, TAIL)], semg.at[0]).wait()
            pltpu.sync_copy(rows_t.at[pl.ds(0, TAIL)],
                            acc.at[didx_t.at[pl.ds(0, TAIL)]], add=True)
        plsc.subcore_barrier()
        pltpu.sync_copy(acc.at[pl.ds(r0, RPW)], stage)
        pltpu.sync_copy(stage, out_hbm.at[c, pl.ds(r0, RPW)])

    return agg_kernel


def _tc1(x, W1, degt, N):
    DH = W1.shape[1]

    def body(x_ref, w_ref, degt_ref, hs_ref, dis_ref):
        degt = degt_ref[...]
        deg = degt[:N, 0:1] + degt[:N, 1:2] + 1.0
        dis = lax.rsqrt(deg)
        h = jnp.dot(x_ref[...], w_ref[...], preferred_element_type=jnp.float32)
        hs_ref[...] = h * dis
        dis_ref[...] = dis

    return pl.pallas_call(
        body,
        out_shape=[
            jax.ShapeDtypeStruct((N, DH), jnp.float32),
            jax.ShapeDtypeStruct((N, 1), jnp.float32),
        ],
    )(x, W1, degt)


def _tc2(aggp, hs, dis, b1, W2p, N):
    DP = W2p.shape[1]

    def body(aggp_ref, hs_ref, dis_ref, b1_ref, w_ref, out_ref):
        agg = aggp_ref[...]
        tot = (agg[0, :N, :] + agg[1, :N, :] + hs_ref[...]) * dis_ref[...]
        z = jnp.maximum(tot + b1_ref[...], 0.0)
        h2 = jnp.dot(z, w_ref[...], preferred_element_type=jnp.float32)
        out_ref[...] = h2 * dis_ref[...]

    return pl.pallas_call(
        body,
        out_shape=jax.ShapeDtypeStruct((N, DP), jnp.float32),
    )(aggp, hs, dis, b1, W2p)


def _tc3(aggp, h2s, dis, b2p, N, DO):
    def body(aggp_ref, h2s_ref, dis_ref, b2_ref, out_ref):
        agg = aggp_ref[...]
        o = (agg[0, :N, :] + agg[1, :N, :] + h2s_ref[...]) * dis_ref[...]
        o = o + b2_ref[...]
        o7 = o[:, :DO]
        m = jnp.max(o7, axis=1, keepdims=True)
        ex = jnp.exp(o7 - m)
        lse = jnp.log(jnp.sum(ex, axis=1, keepdims=True)) + m
        out_ref[...] = o7 - lse

    return pl.pallas_call(
        body,
        out_shape=jax.ShapeDtypeStruct((N, DO), jnp.float32),
    )(aggp, h2s, dis, b2p)


def kernel(x, edge_index, W1, b1, W2, b2):
    N, _ = x.shape
    DH = W1.shape[1]
    DO = W2.shape[1]
    E = edge_index.shape[1]
    DP = 8

    NPAD = -(-N // (NS * 8)) * (NS * 8)

    EPW = E // NW
    NFULL = EPW // CHUNK
    TAIL = EPW - NFULL * CHUNK
    src = edge_index[0].reshape(NW, EPW)
    dst = edge_index[1].reshape(NW, EPW)
    srcm = src[:, :NFULL * CHUNK].reshape(NW, NFULL, CHUNK)
    dstm = dst[:, :NFULL * CHUNK].reshape(NW, NFULL, CHUNK)
    srct = src[:, NFULL * CHUNK:]
    dstt = dst[:, NFULL * CHUNK:]
    ones = jnp.ones((CHUNK,), jnp.float32)
    z1 = jnp.zeros((NPAD,), jnp.float32)
    z16 = jnp.zeros((NPAD, DH), jnp.float32)
    z8 = jnp.zeros((NPAD, DP), jnp.float32)
    W2p = jnp.zeros((DH, DP), jnp.float32).at[:, :DO].set(W2)
    b1r = b1.reshape(1, DH)
    b2p = jnp.zeros((1, DP), jnp.float32).at[0, :DO].set(b2)

    degp = _make_deg(E, NPAD)(dstm, dstt, ones, z1)
    degt = degp.reshape(NC, NPAD).T
    hs, dis = _tc1(x, W1, degt, N)
    aggp1 = _make_agg(E, N, NPAD, DH)(srcm, dstm, srct, dstt, hs, z16)
    h2s = _tc2(aggp1, hs, dis, b1r, W2p, N)
    aggp2 = _make_agg(E, N, NPAD, DP)(srcm, dstm, srct, dstt, h2s, z8)
    return _tc3(aggp2, h2s, dis, b2p, N, DO)


# speedup vs baseline: 81.4151x; 1.3482x over previous
import functools

import jax
import jax.numpy as jnp
from jax import lax
from jax.experimental import pallas as pl
from jax.experimental.pallas import tpu as pltpu
from jax.experimental.pallas import tpu_sc as plsc

NC = 2
NS = 16
NW = NC * NS
CHUNK = 128
DH = 16


def _mesh():
    return plsc.VectorSubcoreMesh(core_axis_name="c", subcore_axis_name="s")


_SC_PARAMS = pltpu.CompilerParams(use_tc_tiling_on_sc=False)


def _group_k(nfull):
    for k in range(16, 0, -1):
        if nfull % k == 0:
            return k
    return 1


def _make_deg(E, NPAD):
    EPW = E // NW
    NFULL = EPW // CHUNK
    TAIL = EPW - NFULL * CHUNK
    RPW = NPAD // NS
    K = _group_k(NFULL)
    NG = NFULL // K

    @functools.partial(
        pl.kernel,
        out_type=jax.ShapeDtypeStruct((NC * NPAD,), jnp.float32),
        mesh=_mesh(),
        compiler_params=_SC_PARAMS,
        scratch_types=[
            pltpu.VMEM((NFULL, CHUNK), jnp.int32),
            pltpu.VMEM((CHUNK,), jnp.float32),
            pltpu.VMEM((max(TAIL, 8),), jnp.int32),
            pltpu.VMEM((max(TAIL, 8),), jnp.float32),
            pltpu.VMEM((RPW,), jnp.float32),
            pltpu.VMEM_SHARED((NPAD,), jnp.float32),
            pltpu.SemaphoreType.DMA,
            pltpu.SemaphoreType.DMA,
        ],
    )
    def deg_kernel(ei_hbm, ones_hbm, zeros_hbm, out_hbm,
                   idx_v, ones_v, idx_t, ones_t, stage, acc, sem, semi):
        c = lax.axis_index("c")
        s = lax.axis_index("s")
        wid = c * NS + s
        r0 = s * RPW

        def pre(j, carry):
            base = wid * EPW + j * CHUNK
            pltpu.async_copy(ei_hbm.at[1, pl.ds(base, CHUNK)],
                             idx_v.at[j], semi)
            return carry

        lax.fori_loop(0, NFULL, pre, 0)
        if TAIL:
            baset = wid * EPW + NFULL * CHUNK
            pltpu.async_copy(ei_hbm.at[1, pl.ds(baset, TAIL)],
                             idx_t.at[pl.ds(0, TAIL)], semi)
        pltpu.sync_copy(zeros_hbm.at[pl.ds(r0, RPW)], stage)
        pltpu.sync_copy(stage, acc.at[pl.ds(r0, RPW)])
        pltpu.sync_copy(ones_hbm, ones_v)
        if TAIL:
            pltpu.sync_copy(ones_hbm.at[pl.ds(0, TAIL)],
                            ones_t.at[pl.ds(0, TAIL)])

        def pre_drain(j, carry):
            base = wid * EPW + j * CHUNK
            pltpu.make_async_copy(ei_hbm.at[1, pl.ds(base, CHUNK)],
                                  idx_v.at[j], semi).wait()
            return carry

        lax.fori_loop(0, NFULL, pre_drain, 0)
        if TAIL:
            pltpu.make_async_copy(ei_hbm.at[1, pl.ds(baset, TAIL)],
                                  idx_t.at[pl.ds(0, TAIL)], semi).wait()
        plsc.subcore_barrier()

        def body(j, carry):
            ds = [pltpu.async_copy(ones_v, acc.at[idx_v.at[j * K + b]], sem,
                                   add=True)
                  for b in range(K)]
            for d in ds:
                d.wait()
            return carry

        lax.fori_loop(0, NG, body, 0)
        if TAIL:
            pltpu.sync_copy(ones_t.at[pl.ds(0, TAIL)],
                            acc.at[idx_t.at[pl.ds(0, TAIL)]], add=True)
        plsc.subcore_barrier()
        pltpu.sync_copy(acc.at[pl.ds(r0, RPW)], stage)
        pltpu.sync_copy(stage, out_hbm.at[pl.ds(c * NPAD + r0, RPW)])

    return deg_kernel


def _make_agg(E, N, NPAD):
    EPW = E // NW
    NFULL = EPW // CHUNK
    TAIL = EPW - NFULL * CHUNK
    RPW = NPAD // NS
    K = _group_k(NFULL)
    NG = NFULL // K
    D = DH

    @functools.partial(
        pl.kernel,
        out_type=jax.ShapeDtypeStruct((NC, NPAD, D), jnp.float32),
        mesh=_mesh(),
        compiler_params=_SC_PARAMS,
        scratch_types=[
            pltpu.VMEM((NFULL, CHUNK), jnp.int32),
            pltpu.VMEM((NFULL, CHUNK), jnp.int32),
            pltpu.VMEM((2, K, CHUNK, D), jnp.float32),
            pltpu.VMEM((max(TAIL, 8),), jnp.int32),
            pltpu.VMEM((max(TAIL, 8),), jnp.int32),
            pltpu.VMEM((max(TAIL, 8), D), jnp.float32),
            pltpu.VMEM((RPW, D), jnp.float32),
            pltpu.VMEM_SHARED((NPAD, D), jnp.float32),
            pltpu.SemaphoreType.DMA((2,)),
            pltpu.SemaphoreType.DMA((2,)),
            pltpu.SemaphoreType.DMA,
        ],
    )
    def agg_kernel(ei_hbm, h_hbm, zeros_hbm, out_hbm,
                   sidx, didx, rows, sidx_t, didx_t, rows_t, stage,
                   acc, semg, sems, semi):
        c = lax.axis_index("c")
        s = lax.axis_index("s")
        wid = c * NS + s
        r0 = s * RPW

        def pre(j, carry):
            base = wid * EPW + j * CHUNK
            pltpu.async_copy(ei_hbm.at[0, pl.ds(base, CHUNK)],
                             sidx.at[j], semi)
            pltpu.async_copy(ei_hbm.at[1, pl.ds(base, CHUNK)],
                             didx.at[j], semi)
            return carry

        lax.fori_loop(0, NFULL, pre, 0)
        if TAIL:
            baset = wid * EPW + NFULL * CHUNK
            pltpu.async_copy(ei_hbm.at[0, pl.ds(baset, TAIL)],
                             sidx_t.at[pl.ds(0, TAIL)], semi)
            pltpu.async_copy(ei_hbm.at[1, pl.ds(baset, TAIL)],
                             didx_t.at[pl.ds(0, TAIL)], semi)
        pltpu.sync_copy(zeros_hbm.at[pl.ds(r0, RPW)], stage)
        pltpu.sync_copy(stage, acc.at[pl.ds(r0, RPW)])

        def pre_drain(j, carry):
            base = wid * EPW + j * CHUNK
            pltpu.make_async_copy(ei_hbm.at[0, pl.ds(base, CHUNK)],
                                  sidx.at[j], semi).wait()
            pltpu.make_async_copy(ei_hbm.at[1, pl.ds(base, CHUNK)],
                                  didx.at[j], semi).wait()
            return carry

        lax.fori_loop(0, NFULL, pre_drain, 0)
        if TAIL:
            pltpu.make_async_copy(ei_hbm.at[0, pl.ds(baset, TAIL)],
                                  sidx_t.at[pl.ds(0, TAIL)], semi).wait()
            pltpu.make_async_copy(ei_hbm.at[1, pl.ds(baset, TAIL)],
                                  didx_t.at[pl.ds(0, TAIL)], semi).wait()
        plsc.subcore_barrier()

        def fire_g(g, h):
            return [pltpu.async_copy(h_hbm.at[sidx.at[g * K + b]],
                                     rows.at[h, b], semg.at[h])
                    for b in range(K)]

        def fire_s(g, h):
            return [pltpu.async_copy(rows.at[h, b],
                                     acc.at[didx.at[g * K + b]],
                                     sems.at[h], add=True)
                    for b in range(K)]

        def drain(ds):
            for d in ds:
                d.wait()

        def drain_g(h):
            for b in range(K):
                pltpu.make_async_copy(h_hbm.at[sidx.at[b]],
                                      rows.at[h, b], semg.at[h]).wait()

        if NG % 2 == 0 and NG >= 2:
            fire_g(0, 0)

            def body(p, carry):
                ga = 2 * p
                drain_g(0)
                sa = fire_s(ga, 0)
                gb = fire_g(ga + 1, 1)
                drain(sa)

                @pl.when(p < NG // 2 - 1)
                def _():
                    fire_g(ga + 2, 0)
                drain(gb)
                drain(fire_s(ga + 1, 1))
                return carry

            lax.fori_loop(0, NG // 2, body, 0)
        else:
            def body1(j, carry):
                drain(fire_g(j, 0))
                drain(fire_s(j, 0))
                return carry

            lax.fori_loop(0, NG, body1, 0)
        if TAIL:
            pltpu.async_copy(h_hbm.at[sidx_t.at[pl.ds(0, TAIL)]],
                             rows_t.at[pl.ds(0, TAIL)], semg.at[0]).wait()
            pltpu.sync_copy(rows_t.at[pl.ds(0, TAIL)],
                            acc.at[didx_t.at[pl.ds(0, TAIL)]], add=True)
        plsc.subcore_barrier()
        pltpu.sync_copy(acc.at[pl.ds(r0, RPW)], stage)
        pltpu.sync_copy(stage, out_hbm.at[c, pl.ds(r0, RPW)])

    return agg_kernel


def _tc1(x, W1, degw, S16, N, NPAD):
    NB = N // 8
    NBP = NPAD // 8

    def body(x_ref, w_ref, degw_ref, s16_ref, hs_ref, dis_ref):
        deg = degw_ref[0] + degw_ref[1] + 1.0
        dis8 = lax.rsqrt(deg)
        dis16 = jnp.dot(dis8, s16_ref[...],
                        preferred_element_type=jnp.float32)
        h = jnp.dot(x_ref[...], w_ref[...],
                    preferred_element_type=jnp.float32)
        hv = h.reshape(NB, 8, DH)
        for s in range(8):
            hs_ref[pl.ds(0, NB), pl.ds(DH * s, DH)] = (
                hv[:, s, :] * dis16[:NB, DH * s:DH * (s + 1)])
        hs_ref[pl.ds(NB, NBP - NB), :] = jnp.zeros(
            (NBP - NB, 128), jnp.float32)
        dis_ref[...] = dis16

    return pl.pallas_call(
        body,
        out_shape=[
            jax.ShapeDtypeStruct((NBP, 128), jnp.float32),
            jax.ShapeDtypeStruct((NBP, 128), jnp.float32),
        ],
    )(x, W1, degw, S16)


def _tc2(aggp, hs, dis16, b1t, W2K, NPAD):
    def body(aggp_ref, hs_ref, dis_ref, b1_ref, w_ref, out_ref):
        agg = aggp_ref[0] + aggp_ref[1]
        d = dis_ref[...]
        tot = (agg + hs_ref[...]) * d + b1_ref[...]
        z = jnp.maximum(tot, 0.0)
        h2 = jnp.dot(z, w_ref[...], preferred_element_type=jnp.float32)
        out_ref[...] = h2 * d

    return pl.pallas_call(
        body,
        out_shape=jax.ShapeDtypeStruct((NPAD // 8, 128), jnp.float32),
    )(aggp, hs, dis16, b1t, W2K)


def _tc3(aggp, h2s, dis16, b2t, N, NPAD, DO):
    NB = N // 8

    def body(aggp_ref, h2s_ref, dis_ref, b2_ref, out_ref):
        agg = aggp_ref[0] + aggp_ref[1]
        d = dis_ref[...]
        o = (agg + h2s_ref[...]) * d + b2_ref[...]
        ov = jnp.stack([o[:NB, DH * s:DH * (s + 1)] for s in range(8)],
                       axis=1)
        ou = ov.reshape(N, DH)
        o7 = ou[:, :DO]
        m = jnp.max(o7, axis=1, keepdims=True)
        ex = jnp.exp(o7 - m)
        lse = jnp.log(jnp.sum(ex, axis=1, keepdims=True)) + m
        out_ref[...] = o7 - lse

    return pl.pallas_call(
        body,
        out_shape=jax.ShapeDtypeStruct((N, DO), jnp.float32),
    )(aggp, h2s, dis16, b2t)


def kernel(x, edge_index, W1, b1, W2, b2):
    N, _ = x.shape
    DO = W2.shape[1]
    E = edge_index.shape[1]

    NPAD = -(-N // (NS * 8)) * (NS * 8)

    eye8 = jnp.eye(8, dtype=jnp.float32)
    W2p = jnp.zeros((DH, DH), jnp.float32).at[:, :DO].set(W2)
    W2K = jnp.kron(eye8, W2p)
    S16 = jnp.kron(eye8, jnp.ones((1, DH), jnp.float32))
    b1t = jnp.tile(b1, 8).reshape(1, 128)
    b2t = jnp.tile(jnp.concatenate(
        [b2, jnp.zeros((DH - DO,), jnp.float32)]), 8).reshape(1, 128)

    ones = jnp.ones((CHUNK,), jnp.float32)
    z1 = jnp.zeros((NPAD,), jnp.float32)
    z16 = jnp.zeros((NPAD, DH), jnp.float32)

    degp = _make_deg(E, NPAD)(edge_index, ones, z1)
    degw = degp.reshape(NC, NPAD // 8, 8)
    hsP, dis16 = _tc1(x, W1, degw, S16, N, NPAD)
    hs = hsP.reshape(NPAD, DH)
    aggp1 = _make_agg(E, N, NPAD)(edge_index, hs, z16)
    aggp1P = aggp1.reshape(NC, NPAD // 8, 128)
    h2sP = _tc2(aggp1P, hsP, dis16, b1t, W2K, NPAD)
    h2s = h2sP.reshape(NPAD, DH)
    aggp2 = _make_agg(E, N, NPAD)(edge_index, h2s, z16)
    aggp2P = aggp2.reshape(NC, NPAD // 8, 128)
    return _tc3(aggp2P, h2sP, dis16, b2t, N, NPAD, DO)


# speedup vs baseline: 92.2255x; 1.1328x over previous
import functools

import jax
import jax.numpy as jnp
from jax import lax
from jax.experimental import pallas as pl
from jax.experimental.pallas import tpu as pltpu
from jax.experimental.pallas import tpu_sc as plsc

NC = 2
NS = 16
NW = NC * NS
CHUNK = 128
DH = 16


def _mesh():
    return plsc.VectorSubcoreMesh(core_axis_name="c", subcore_axis_name="s")


_SC_PARAMS = pltpu.CompilerParams(use_tc_tiling_on_sc=False)


def _group_k(nfull):
    for k in range(16, 0, -1):
        if nfull % k == 0:
            return k
    return 1


def _make_deg(E, NPAD):
    EPW = E // NW
    NFULL = EPW // CHUNK
    TAIL = EPW - NFULL * CHUNK
    RPW = NPAD // NS
    K = _group_k(NFULL)
    NG = NFULL // K

    @functools.partial(
        pl.kernel,
        out_type=jax.ShapeDtypeStruct((NC * NPAD,), jnp.float32),
        mesh=_mesh(),
        compiler_params=_SC_PARAMS,
        scratch_types=[
            pltpu.VMEM((NFULL, CHUNK), jnp.int32),
            pltpu.VMEM((CHUNK,), jnp.float32),
            pltpu.VMEM((max(TAIL, 8),), jnp.int32),
            pltpu.VMEM((max(TAIL, 8),), jnp.float32),
            pltpu.VMEM((RPW,), jnp.float32),
            pltpu.VMEM_SHARED((NPAD,), jnp.float32),
            pltpu.SemaphoreType.DMA,
            pltpu.SemaphoreType.DMA,
        ],
    )
    def deg_kernel(ei_hbm, ones_hbm, zeros_hbm, out_hbm,
                   idx_v, ones_v, idx_t, ones_t, stage, acc, sem, semi):
        c = lax.axis_index("c")
        s = lax.axis_index("s")
        wid = c * NS + s
        r0 = s * RPW

        def pre(j, carry):
            base = wid * EPW + j * CHUNK
            pltpu.async_copy(ei_hbm.at[1, pl.ds(base, CHUNK)],
                             idx_v.at[j], semi)
            return carry

        lax.fori_loop(0, NFULL, pre, 0)
        if TAIL:
            baset = wid * EPW + NFULL * CHUNK
            pltpu.async_copy(ei_hbm.at[1, pl.ds(baset, TAIL)],
                             idx_t.at[pl.ds(0, TAIL)], semi)
        pltpu.sync_copy(zeros_hbm.at[pl.ds(r0, RPW)], stage)
        pltpu.sync_copy(stage, acc.at[pl.ds(r0, RPW)])
        pltpu.sync_copy(ones_hbm, ones_v)
        if TAIL:
            pltpu.sync_copy(ones_hbm.at[pl.ds(0, TAIL)],
                            ones_t.at[pl.ds(0, TAIL)])

        def pre_drain(j, carry):
            base = wid * EPW + j * CHUNK
            pltpu.make_async_copy(ei_hbm.at[1, pl.ds(base, CHUNK)],
                                  idx_v.at[j], semi).wait()
            return carry

        lax.fori_loop(0, NFULL, pre_drain, 0)
        if TAIL:
            pltpu.make_async_copy(ei_hbm.at[1, pl.ds(baset, TAIL)],
                                  idx_t.at[pl.ds(0, TAIL)], semi).wait()
        plsc.subcore_barrier()

        def body(j, carry):
            ds = [pltpu.async_copy(ones_v, acc.at[idx_v.at[j * K + b]], sem,
                                   add=True)
                  for b in range(K)]
            for d in ds:
                d.wait()
            return carry

        lax.fori_loop(0, NG, body, 0)
        if TAIL:
            pltpu.sync_copy(ones_t.at[pl.ds(0, TAIL)],
                            acc.at[idx_t.at[pl.ds(0, TAIL)]], add=True)
        plsc.subcore_barrier()
        pltpu.sync_copy(acc.at[pl.ds(r0, RPW)], stage)
        pltpu.sync_copy(stage, out_hbm.at[pl.ds(c * NPAD + r0, RPW)])

    return deg_kernel


def _make_agg(E, N, NPAD):
    EPW = E // NW
    NFULL = EPW // CHUNK
    TAIL = EPW - NFULL * CHUNK
    RPW = NPAD // NS
    K = _group_k(NFULL)
    NG = NFULL // K
    D = DH

    @functools.partial(
        pl.kernel,
        out_type=jax.ShapeDtypeStruct((NC, NPAD, D), jnp.float32),
        mesh=_mesh(),
        compiler_params=_SC_PARAMS,
        scratch_types=[
            pltpu.VMEM((NFULL, CHUNK), jnp.int32),
            pltpu.VMEM((NFULL, CHUNK), jnp.int32),
            pltpu.VMEM((2, K, CHUNK, D), jnp.float32),
            pltpu.VMEM((max(TAIL, 8),), jnp.int32),
            pltpu.VMEM((max(TAIL, 8),), jnp.int32),
            pltpu.VMEM((max(TAIL, 8), D), jnp.float32),
            pltpu.VMEM((RPW, D), jnp.float32),
            pltpu.VMEM_SHARED((NPAD, D), jnp.float32),
            pltpu.SemaphoreType.DMA((2,)),
            pltpu.SemaphoreType.DMA((2,)),
            pltpu.SemaphoreType.DMA,
        ],
    )
    def agg_kernel(ei_hbm, h_hbm, zeros_hbm, out_hbm,
                   sidx, didx, rows, sidx_t, didx_t, rows_t, stage,
                   acc, semg, sems, semi):
        c = lax.axis_index("c")
        s = lax.axis_index("s")
        wid = c * NS + s
        r0 = s * RPW

        def pre(j, carry):
            base = wid * EPW + j * CHUNK
            pltpu.async_copy(ei_hbm.at[0, pl.ds(base, CHUNK)],
                             sidx.at[j], semi)
            pltpu.async_copy(ei_hbm.at[1, pl.ds(base, CHUNK)],
                             didx.at[j], semi)
            return carry

        lax.fori_loop(0, NFULL, pre, 0)
        if TAIL:
            baset = wid * EPW + NFULL * CHUNK
            pltpu.async_copy(ei_hbm.at[0, pl.ds(baset, TAIL)],
                             sidx_t.at[pl.ds(0, TAIL)], semi)
            pltpu.async_copy(ei_hbm.at[1, pl.ds(baset, TAIL)],
                             didx_t.at[pl.ds(0, TAIL)], semi)
        pltpu.sync_copy(zeros_hbm.at[pl.ds(r0, RPW)], stage)
        pltpu.sync_copy(stage, acc.at[pl.ds(r0, RPW)])

        def pre_drain(j, carry):
            base = wid * EPW + j * CHUNK
            pltpu.make_async_copy(ei_hbm.at[0, pl.ds(base, CHUNK)],
                                  sidx.at[j], semi).wait()
            pltpu.make_async_copy(ei_hbm.at[1, pl.ds(base, CHUNK)],
                                  didx.at[j], semi).wait()
            return carry

        lax.fori_loop(0, NFULL, pre_drain, 0)
        if TAIL:
            pltpu.make_async_copy(ei_hbm.at[0, pl.ds(baset, TAIL)],
                                  sidx_t.at[pl.ds(0, TAIL)], semi).wait()
            pltpu.make_async_copy(ei_hbm.at[1, pl.ds(baset, TAIL)],
                                  didx_t.at[pl.ds(0, TAIL)], semi).wait()
        plsc.subcore_barrier()

        def fire_g(g, h):
            return [pltpu.async_copy(h_hbm.at[sidx.at[g * K + b]],
                                     rows.at[h, b], semg.at[h])
                    for b in range(K)]

        def fire_s(g, h):
            return [pltpu.async_copy(rows.at[h, b],
                                     acc.at[didx.at[g * K + b]],
                                     sems.at[h], add=True)
                    for b in range(K)]

        def drain(ds):
            for d in ds:
                d.wait()

        def drain_g(h):
            for b in range(K):
                pltpu.make_async_copy(h_hbm.at[sidx.at[b]],
                                      rows.at[h, b], semg.at[h]).wait()

        if NG % 2 == 0 and NG >= 2:
            fire_g(0, 0)

            def body(p, carry):
                ga = 2 * p
                drain_g(0)
                sa = fire_s(ga, 0)
                gb = fire_g(ga + 1, 1)
                drain(sa)

                @pl.when(p < NG // 2 - 1)
                def _():
                    fire_g(ga + 2, 0)
                drain(gb)
                drain(fire_s(ga + 1, 1))
                return carry

            lax.fori_loop(0, NG // 2, body, 0)
        else:
            def body1(j, carry):
                drain(fire_g(j, 0))
                drain(fire_s(j, 0))
                return carry

            lax.fori_loop(0, NG, body1, 0)
        if TAIL:
            pltpu.async_copy(h_hbm.at[sidx_t.at[pl.ds(0, TAIL)]],
                             rows_t.at[pl.ds(0, TAIL)], semg.at[0]).wait()
            pltpu.sync_copy(rows_t.at[pl.ds(0, TAIL)],
                            acc.at[didx_t.at[pl.ds(0, TAIL)]], add=True)
        plsc.subcore_barrier()
        pltpu.sync_copy(acc.at[pl.ds(r0, RPW)], stage)
        pltpu.sync_copy(stage, out_hbm.at[c, pl.ds(r0, RPW)])

    return agg_kernel


def _tc1(x, W1, degw, S16, N, NPAD):
    NB = N // 8
    NBP = NPAD // 8

    def body(x_ref, w_ref, degw_ref, s16_ref, hs_ref, dis_ref):
        deg = degw_ref[0] + degw_ref[1] + 1.0
        dis8 = lax.rsqrt(deg)
        dis16 = jnp.dot(dis8, s16_ref[...],
                        preferred_element_type=jnp.float32)
        h16 = jnp.dot(x_ref[...], w_ref[...],
                      preferred_element_type=jnp.float32)
        hv = h16.reshape(NB, 8, 128)
        for s in range(8):
            sl = slice(DH * s, DH * (s + 1))
            hs_ref[pl.ds(0, NB), pl.ds(DH * s, DH)] = (
                hv[:, s, sl] * dis16[:NB, sl])
        hs_ref[pl.ds(NB, NBP - NB), :] = jnp.zeros(
            (NBP - NB, 128), jnp.float32)
        dis_ref[...] = dis16

    return pl.pallas_call(
        body,
        out_shape=[
            jax.ShapeDtypeStruct((NBP, 128), jnp.float32),
            jax.ShapeDtypeStruct((NBP, 128), jnp.float32),
        ],
    )(x, W1, degw, S16)


def _tc2(aggp, hs, dis16, b1t, W2K, NPAD):
    def body(aggp_ref, hs_ref, dis_ref, b1_ref, w_ref, out_ref):
        agg = aggp_ref[0] + aggp_ref[1]
        d = dis_ref[...]
        tot = (agg + hs_ref[...]) * d + b1_ref[...]
        z = jnp.maximum(tot, 0.0)
        h2 = jnp.dot(z, w_ref[...], preferred_element_type=jnp.float32)
        out_ref[...] = h2 * d

    return pl.pallas_call(
        body,
        out_shape=jax.ShapeDtypeStruct((NPAD // 8, 128), jnp.float32),
    )(aggp, hs, dis16, b1t, W2K)


def _tc3(aggp, h2s, dis16, b2t, shifts, maxb, sumb, NPAD, DO):
    OFF = 2.0e6

    def body(aggp_ref, h2s_ref, dis_ref, b2_ref, sh_ref, mb_ref, sb_ref,
             out_ref):
        agg = aggp_ref[0] + aggp_ref[1]
        d = dis_ref[...]
        o = (agg + h2s_ref[...]) * d + b2_ref[...]
        lane = lax.broadcasted_iota(jnp.int32, o.shape, 1)
        valid = (lane & 15) < DO
        y = jnp.where(valid, o + OFF, 0.0)
        m = y
        for k in range(4):
            m = jnp.maximum(m, jnp.dot(m, sh_ref[k],
                                       preferred_element_type=jnp.float32))
        c = jnp.dot(m, mb_ref[...],
                    preferred_element_type=jnp.float32) - OFF
        e = jnp.where(valid, jnp.exp(o - c), 0.0)
        s = jnp.dot(e, sb_ref[...], preferred_element_type=jnp.float32)
        out_ref[...] = o - c - jnp.log(s)

    return pl.pallas_call(
        body,
        out_shape=jax.ShapeDtypeStruct((NPAD // 8, 128), jnp.float32),
    )(aggp, h2s, dis16, b2t, shifts, maxb, sumb)


def kernel(x, edge_index, W1, b1, W2, b2):
    N, _ = x.shape
    DO = W2.shape[1]
    E = edge_index.shape[1]

    NPAD = -(-N // (NS * 8)) * (NS * 8)

    eye8 = jnp.eye(8, dtype=jnp.float32)
    W1r = jnp.tile(W1, (1, 8))
    W2p = jnp.zeros((DH, DH), jnp.float32).at[:, :DO].set(W2)
    W2K = jnp.kron(eye8, W2p)
    S16 = jnp.kron(eye8, jnp.ones((1, DH), jnp.float32))
    shifts = jnp.stack([
        jnp.kron(eye8, jnp.eye(DH, k=-k, dtype=jnp.float32))
        for k in (1, 2, 4, 8)])
    maxb = jnp.kron(
        eye8, jnp.zeros((DH, DH), jnp.float32).at[0, :].set(1.0))
    sumb = jnp.kron(eye8, jnp.ones((DH, DH), jnp.float32))
    b1t = jnp.tile(b1, 8).reshape(1, 128)
    b2t = jnp.tile(jnp.concatenate(
        [b2, jnp.zeros((DH - DO,), jnp.float32)]), 8).reshape(1, 128)

    ones = jnp.ones((CHUNK,), jnp.float32)
    z1 = jnp.zeros((NPAD,), jnp.float32)
    z16 = jnp.zeros((NPAD, DH), jnp.float32)

    degp = _make_deg(E, NPAD)(edge_index, ones, z1)
    degw = degp.reshape(NC, NPAD // 8, 8)
    hsP, dis16 = _tc1(x, W1r, degw, S16, N, NPAD)
    hs = hsP.reshape(NPAD, DH)
    aggp1 = _make_agg(E, N, NPAD)(edge_index, hs, z16)
    aggp1P = aggp1.reshape(NC, NPAD // 8, 128)
    h2sP = _tc2(aggp1P, hsP, dis16, b1t, W2K, NPAD)
    h2s = h2sP.reshape(NPAD, DH)
    aggp2 = _make_agg(E, N, NPAD)(edge_index, h2s, z16)
    aggp2P = aggp2.reshape(NC, NPAD // 8, 128)
    lsmP = _tc3(aggp2P, h2sP, dis16, b2t, shifts, maxb, sumb, NPAD, DO)
    return lsmP.reshape(NPAD, DH)[:N, :DO]
